# Initial kernel scaffold; baseline (speedup 1.0000x reference)
import jax
import jax.numpy as jnp
from jax.experimental import pallas as pl


def kernel(x, edge_index, batch, W1_0, b1_0, W2_0, b2_0, W1_1, b1_1, W2_1, b2_1, W1_2, b1_2, W2_2, b2_2, gamma0, beta0, gamma1, beta1, Wp1, bp1, Wp2, bp2):
    raise NotImplementedError("write your pallas kernel here")


# speedup vs baseline: 4.5410x; 4.5410x over previous
import functools

import jax
import jax.numpy as jnp
from jax import lax
from jax.experimental import pallas as pl
from jax.experimental.pallas import tpu as pltpu
from jax.experimental.pallas import tpu_sc as plsc

_N = 10000
_D = 128
_E = 320000
_G = 64

_NC = 2
_NS = 16
_NW = _NC * _NS
_EPW = _E // _NW
_CH = 80
_NCHUNK = _EPW // _CH
_NACC = 10240
_RPT = _NACC // _NS

_R = 2000
_NB = _N // _R


def _sc_agg(h, src, dst, zeros):
    mesh = plsc.VectorSubcoreMesh(
        core_axis_name="c", subcore_axis_name="s",
        num_cores=_NC, num_subcores=_NS)

    @functools.partial(
        pl.kernel,
        out_type=jax.ShapeDtypeStruct((_NC, _NACC, _D), jnp.float32),
        mesh=mesh,
        scratch_types=[
            pltpu.VMEM((_CH,), jnp.int32),
            pltpu.VMEM((_CH,), jnp.int32),
            pltpu.VMEM((_CH, _D), jnp.float32),
            pltpu.VMEM_SHARED((_NACC, _D), jnp.float32),
            pltpu.SemaphoreType.DMA,
        ],
    )
    def k(h_hbm, src_hbm, dst_hbm, z_hbm, out_hbm, src_v, dst_v, rows_v, acc, sem):
        c = lax.axis_index("c")
        s = lax.axis_index("s")
        wid = s * _NC + c
        row0 = s * _RPT
        pltpu.sync_copy(z_hbm.at[pl.ds(row0, _RPT)], acc.at[pl.ds(row0, _RPT)])
        plsc.subcore_barrier()
        e0 = wid * _EPW

        def body(i, carry):
            base = e0 + i * _CH
            pltpu.sync_copy(src_hbm.at[pl.ds(base, _CH)], src_v)
            pltpu.sync_copy(dst_hbm.at[pl.ds(base, _CH)], dst_v)
            pltpu.async_copy(h_hbm.at[src_v], rows_v, sem).wait()
            pltpu.sync_copy(rows_v, acc.at[dst_v], add=True)
            return carry

        lax.fori_loop(0, _NCHUNK, body, 0)
        plsc.subcore_barrier()
        pltpu.sync_copy(acc.at[pl.ds(row0, _RPT)],
                        out_hbm.at[c, pl.ds(row0, _RPT)])

    return k(h, src, dst, zeros)


def _tc_layer(h, p, W1, b1, W2, b2, gamma, beta):
    def body(h_ref, p0_ref, p1_ref, w1_ref, b1_ref, w2_ref, b2_ref,
             g_ref, be_ref, o_ref):
        t = h_ref[...] + p0_ref[0] + p1_ref[0]
        u = jnp.dot(t, w1_ref[...], preferred_element_type=jnp.float32,
                    precision=lax.Precision.HIGHEST) + b1_ref[...]
        u = jnp.maximum(u, 0.0)
        v = jnp.dot(u, w2_ref[...], preferred_element_type=jnp.float32,
                    precision=lax.Precision.HIGHEST) + b2_ref[...]
        r = jnp.maximum(v, 0.0)
        mu = jnp.mean(r, axis=1, keepdims=True)
        var = jnp.mean((r - mu) ** 2, axis=1, keepdims=True)
        o_ref[...] = (r - mu) * lax.rsqrt(var + 1e-5) * g_ref[...] + be_ref[...]

    full = pl.BlockSpec((_D, _D), lambda i: (0, 0))
    vec = pl.BlockSpec((1, _D), lambda i: (0, 0))
    return pl.pallas_call(
        body,
        grid=(_NB,),
        in_specs=[
            pl.BlockSpec((_R, _D), lambda i: (i, 0)),
            pl.BlockSpec((1, _R, _D), lambda i: (0, i, 0)),
            pl.BlockSpec((1, _R, _D), lambda i: (1, i, 0)),
            full, vec, full, vec, vec, vec,
        ],
        out_specs=pl.BlockSpec((_R, _D), lambda i: (i, 0)),
        out_shape=jax.ShapeDtypeStruct((_N, _D), jnp.float32),
    )(h, p, p, W1, b1, W2, b2, gamma, beta)


def _tc_final(h, p, batch_r, W1, b1, W2, b2, Wp1, bp1, Wp2p, bp2p):
    def body(h_ref, p0_ref, p1_ref, w1_ref, b1_ref, w2_ref, b2_ref,
             bat_ref, wp1_ref, bp1_ref, wp2_ref, bp2_ref,
             emb_ref, out_ref, sums, counts):
        i = pl.program_id(0)

        @pl.when(i == 0)
        def _init():
            sums[...] = jnp.zeros((_G, _D), jnp.float32)
            counts[...] = jnp.zeros((_G, _D), jnp.float32)

        t = h_ref[...] + p0_ref[0] + p1_ref[0]
        u = jnp.dot(t, w1_ref[...], preferred_element_type=jnp.float32,
                    precision=lax.Precision.HIGHEST) + b1_ref[...]
        u = jnp.maximum(u, 0.0)
        v = jnp.dot(u, w2_ref[...], preferred_element_type=jnp.float32,
                    precision=lax.Precision.HIGHEST) + b2_ref[...]
        r = jnp.maximum(v, 0.0)

        bidx = bat_ref[0, 0, :]
        gids = lax.broadcasted_iota(jnp.int32, (_R, _G), 1)
        onehot = (bidx[:, None] == gids).astype(jnp.float32)
        sums[...] += lax.dot_general(
            onehot, r, (((0,), (0,)), ((), ())),
            preferred_element_type=jnp.float32,
            precision=lax.Precision.HIGHEST)
        counts[...] += jnp.sum(onehot, axis=0)[:, None]

        @pl.when(i == _NB - 1)
        def _fin():
            pooled = sums[...] / jnp.maximum(counts[...], 1.0)
            emb_ref[...] = pooled
            z = jnp.dot(pooled, wp1_ref[...],
                        preferred_element_type=jnp.float32,
                        precision=lax.Precision.HIGHEST) + bp1_ref[...]
            z2 = jnp.dot(z, wp2_ref[...],
                         preferred_element_type=jnp.float32,
                         precision=lax.Precision.HIGHEST) + bp2_ref[...]
            col = lax.broadcasted_iota(jnp.int32, (_G, _D), 1)
            valid = col < 2
            zm = jnp.where(valid, z2, -jnp.inf)
            m = jnp.max(zm, axis=1, keepdims=True)
            lse = m + jnp.log(jnp.sum(
                jnp.where(valid, jnp.exp(zm - m), 0.0),
                axis=1, keepdims=True))
            out_ref[...] = jnp.where(valid, z2 - lse, 0.0)

    full = pl.BlockSpec((_D, _D), lambda i: (0, 0))
    vec = pl.BlockSpec((1, _D), lambda i: (0, 0))
    return pl.pallas_call(
        body,
        grid=(_NB,),
        in_specs=[
            pl.BlockSpec((_R, _D), lambda i: (i, 0)),
            pl.BlockSpec((1, _R, _D), lambda i: (0, i, 0)),
            pl.BlockSpec((1, _R, _D), lambda i: (1, i, 0)),
            full, vec, full, vec,
            pl.BlockSpec((1, 1, _R), lambda i: (i, 0, 0)),
            full, vec, full, vec,
        ],
        out_specs=[
            pl.BlockSpec((_G, _D), lambda i: (0, 0)),
            pl.BlockSpec((_G, _D), lambda i: (0, 0)),
        ],
        out_shape=[
            jax.ShapeDtypeStruct((_G, _D), jnp.float32),
            jax.ShapeDtypeStruct((_G, _D), jnp.float32),
        ],
        scratch_shapes=[
            pltpu.VMEM((_G, _D), jnp.float32),
            pltpu.VMEM((_G, _D), jnp.float32),
        ],
    )(h, p, p, W1, b1, W2, b2, batch_r, Wp1, bp1, Wp2p, bp2p)


def kernel(x, edge_index, batch, W1_0, b1_0, W2_0, b2_0, W1_1, b1_1, W2_1,
           b2_1, W1_2, b1_2, W2_2, b2_2, gamma0, beta0, gamma1, beta1,
           Wp1, bp1, Wp2, bp2):
    src = edge_index[0]
    dst = edge_index[1]
    zeros = jnp.zeros((_NACC, _D), jnp.float32)
    batch_r = batch.reshape(_NB, 1, _R)

    r2 = lambda a: a.reshape(1, _D)
    Wp2p = jnp.zeros((_D, _D), jnp.float32).at[:, :2].set(Wp2)
    bp2p = jnp.zeros((1, _D), jnp.float32).at[0, :2].set(bp2)

    p = _sc_agg(x, src, dst, zeros)
    h = _tc_layer(x, p, W1_0, r2(b1_0), W2_0, r2(b2_0), r2(gamma0), r2(beta0))
    p = _sc_agg(h, src, dst, zeros)
    h = _tc_layer(h, p, W1_1, r2(b1_1), W2_1, r2(b2_1), r2(gamma1), r2(beta1))
    p = _sc_agg(h, src, dst, zeros)
    emb, outp = _tc_final(h, p, batch_r, W1_2, r2(b1_2), W2_2, r2(b2_2),
                          Wp1, r2(bp1), Wp2p, bp2p)
    return (emb, outp[:, :2])
